# baseline (speedup 1.0000x reference)
import functools

import jax
import jax.numpy as jnp
from jax import lax
from jax.experimental import pallas as pl
from jax.experimental.pallas import tpu as pltpu
from jax.experimental.pallas import tpu_sc as plsc

_NC = 2
_NS = 16
_K = 80
_PH = 64


def _sc_segment_sum(x, e4, zeros_pad):
    n, d = x.shape
    nblk = e4.shape[2]
    n_pad = zeros_pad.shape[0]
    rows_per_sub = n_pad // _NS

    mesh = plsc.VectorSubcoreMesh(core_axis_name="c", subcore_axis_name="s")

    nbuf = 2
    n_phase = nblk // _PH
    assert nblk == n_phase * _PH and _PH % 2 == 0 and _PH >= 4

    @functools.partial(
        pl.kernel,
        out_type=jax.ShapeDtypeStruct((_NC, n_pad, d), jnp.float32),
        mesh=mesh,
        scratch_types=[
            pltpu.VMEM_SHARED((n_pad, d), jnp.float32),
            pltpu.VMEM((_PH, _K), jnp.int32),
            pltpu.VMEM((_PH, _K), jnp.int32),
            pltpu.VMEM((_K, d), jnp.float32),
            pltpu.SemaphoreType.DMA((nbuf,)),
            pltpu.SemaphoreType.DMA((nbuf,)),
        ],
    )
    def k(x_hbm, e_hbm, z_hbm, o_hbm, acc, sidx, didx, rows, semg, sems):
        c = lax.axis_index("c")
        s = lax.axis_index("s")
        wid = c * _NS + s

        r0 = s * rows_per_sub
        pltpu.sync_copy(z_hbm.at[pl.ds(r0, rows_per_sub)],
                        acc.at[pl.ds(r0, rows_per_sub)])
        plsc.subcore_barrier()

        def g_start(b, i):
            pltpu.async_copy(x_hbm.at[sidx.at[b]], rows.at[i], semg.at[i])

        def g_wait(b, i):
            pltpu.make_async_copy(
                x_hbm.at[sidx.at[b]], rows.at[i], semg.at[i]).wait()

        def s_start(b, i):
            pltpu.async_copy(rows.at[i], acc.at[didx.at[b]], sems.at[i],
                             add=True)

        def s_wait(b, i):
            pltpu.make_async_copy(
                rows.at[i], acc.at[didx.at[b]], sems.at[i]).wait()

        for p in range(n_phase):
            pltpu.sync_copy(e_hbm.at[0, wid, pl.ds(p * _PH, _PH)], sidx)
            pltpu.sync_copy(e_hbm.at[1, wid, pl.ds(p * _PH, _PH)], didx)

            @pl.loop(0, _PH)
            def _(b):
                pltpu.sync_copy(x_hbm.at[sidx.at[b]], rows)
                pltpu.sync_copy(rows, acc.at[didx.at[b]], add=True)

        plsc.subcore_barrier()
        pltpu.sync_copy(acc.at[pl.ds(r0, rows_per_sub)],
                        o_hbm.at[c, pl.ds(r0, rows_per_sub)])

    return k(x, e4, zeros_pad)


def _tc_linear(x, agg, w, b, relu):
    n, d = x.shape
    h = w.shape[1]
    br = 1000

    def kern(x_ref, a_ref, w_ref, b_ref, o_ref):
        t = x_ref[...] + a_ref[0] + a_ref[1]
        y = jnp.dot(t, w_ref[...], preferred_element_type=jnp.float32)
        y = y + b_ref[...]
        o_ref[...] = jnp.maximum(y, 0.0) if relu else y

    return pl.pallas_call(
        kern,
        grid=(n // br,),
        in_specs=[
            pl.BlockSpec((br, d), lambda i: (i, 0)),
            pl.BlockSpec((_NC, br, d), lambda i: (0, i, 0)),
            pl.BlockSpec((d, h), lambda i: (0, 0)),
            pl.BlockSpec((1, h), lambda i: (0, 0)),
        ],
        out_specs=pl.BlockSpec((br, h), lambda i: (i, 0)),
        out_shape=jax.ShapeDtypeStruct((n, h), jnp.float32),
    )(x, agg, w, b.reshape(1, h))


def kernel(features, edge_index, W1, b1, W2, b2):
    n, d = features.shape
    e = edge_index.shape[1]
    c = W2.shape[1]
    n_workers = _NC * _NS
    n_per = e // n_workers
    nblk = -(-(-(-n_per // _K)) // _PH) * _PH
    n_pad = -(-n // 128) * 128

    e3 = edge_index.reshape(2, n_workers, n_per)
    padlen = nblk * _K - n_per
    if padlen:
        scrap = n + (jnp.arange(n_workers * padlen, dtype=jnp.int32)
                     % (n_pad - n)).reshape(n_workers, padlen)
        fill = jnp.stack([jnp.zeros((n_workers, padlen), jnp.int32), scrap])
        e3 = jnp.concatenate([e3, fill], axis=2)
    e4 = e3.reshape(2, n_workers, nblk, _K)
    zeros_pad = jnp.zeros((n_pad, d), jnp.float32)

    agg1 = _sc_segment_sum(features, e4, zeros_pad)
    x = _tc_linear(features, agg1[:, :n], W1, b1, relu=True)
    agg2 = _sc_segment_sum(x, e4, zeros_pad)

    c_pad = -(-c // 128) * 128
    w2p = jnp.pad(W2, ((0, 0), (0, c_pad - c)))
    b2p = jnp.pad(b2, (0, c_pad - c))
    logits = _tc_linear(x, agg2[:, :n], w2p, b2p, relu=False)
    return logits[:, :c]


# speedup vs baseline: 1.0020x; 1.0020x over previous
import functools

import jax
import jax.numpy as jnp
from jax import lax
from jax.experimental import pallas as pl
from jax.experimental.pallas import tpu as pltpu
from jax.experimental.pallas import tpu_sc as plsc

_NC = 2
_NS = 16
_K = 80
_PH = 128


def _sc_segment_sum(x, e4, zeros_pad):
    n, d = x.shape
    nblk = e4.shape[2]
    n_pad = zeros_pad.shape[0]
    rows_per_sub = n_pad // _NS
    assert nblk % 2 == 0 and nblk >= 4

    mesh = plsc.VectorSubcoreMesh(core_axis_name="c", subcore_axis_name="s")

    @functools.partial(
        pl.kernel,
        out_type=jax.ShapeDtypeStruct((_NC, n_pad, d), jnp.float32),
        mesh=mesh,
        scratch_types=[
            pltpu.VMEM_SHARED((n_pad, d), jnp.float32),
            pltpu.VMEM((nblk, _K), jnp.int32),
            pltpu.VMEM((nblk, _K), jnp.int32),
            pltpu.VMEM((_K, d), jnp.float32),
            pltpu.VMEM((_K, d), jnp.float32),
            pltpu.SemaphoreType.DMA((2,)),
            pltpu.SemaphoreType.DMA((2,)),
        ],
    )
    def k(x_hbm, e_hbm, z_hbm, o_hbm, acc, sidx, didx, rows0, rows1,
          semg, sems):
        c = lax.axis_index("c")
        s = lax.axis_index("s")
        wid = c * _NS + s
        rbuf = (rows0, rows1)

        r0 = s * rows_per_sub
        pltpu.sync_copy(z_hbm.at[pl.ds(r0, rows_per_sub)],
                        acc.at[pl.ds(r0, rows_per_sub)])
        pltpu.sync_copy(e_hbm.at[0, wid], sidx)
        pltpu.sync_copy(e_hbm.at[1, wid], didx)
        plsc.subcore_barrier()

        @pl.loop(0, nblk)
        def _(b):
            pltpu.sync_copy(x_hbm.at[sidx.at[b]], rows0)
            pltpu.sync_copy(rows0, acc.at[didx.at[b]], add=True)

        plsc.subcore_barrier()
        pltpu.sync_copy(acc.at[pl.ds(r0, rows_per_sub)],
                        o_hbm.at[c, pl.ds(r0, rows_per_sub)])

    return k(x, e4, zeros_pad)


def _tc_linear(x, agg, w, b, relu):
    n, d = x.shape
    h = w.shape[1]
    br = 1000

    def kern(x_ref, a_ref, w_ref, b_ref, o_ref):
        t = x_ref[...] + a_ref[0] + a_ref[1]
        y = jnp.dot(t, w_ref[...], preferred_element_type=jnp.float32)
        y = y + b_ref[...]
        o_ref[...] = jnp.maximum(y, 0.0) if relu else y

    return pl.pallas_call(
        kern,
        grid=(n // br,),
        in_specs=[
            pl.BlockSpec((br, d), lambda i: (i, 0)),
            pl.BlockSpec((_NC, br, d), lambda i: (0, i, 0)),
            pl.BlockSpec((d, h), lambda i: (0, 0)),
            pl.BlockSpec((1, h), lambda i: (0, 0)),
        ],
        out_specs=pl.BlockSpec((br, h), lambda i: (i, 0)),
        out_shape=jax.ShapeDtypeStruct((n, h), jnp.float32),
    )(x, agg, w, b.reshape(1, h))


def kernel(features, edge_index, W1, b1, W2, b2):
    n, d = features.shape
    e = edge_index.shape[1]
    c = W2.shape[1]
    n_workers = _NC * _NS
    n_per = e // n_workers
    nblk = -(-(-(-n_per // _K)) // _PH) * _PH
    n_pad = -(-n // 128) * 128

    e3 = edge_index.reshape(2, n_workers, n_per)
    padlen = nblk * _K - n_per
    if padlen:
        scrap = n + (jnp.arange(n_workers * padlen, dtype=jnp.int32)
                     % (n_pad - n)).reshape(n_workers, padlen)
        fill = jnp.stack([jnp.zeros((n_workers, padlen), jnp.int32), scrap])
        e3 = jnp.concatenate([e3, fill], axis=2)
    e4 = e3.reshape(2, n_workers, nblk, _K)
    zeros_pad = jnp.zeros((n_pad, d), jnp.float32)

    agg1 = _sc_segment_sum(features, e4, zeros_pad)
    x = _tc_linear(features, agg1[:, :n], W1, b1, relu=True)
    agg2 = _sc_segment_sum(x, e4, zeros_pad)

    c_pad = -(-c // 128) * 128
    w2p = jnp.pad(W2, ((0, 0), (0, c_pad - c)))
    b2p = jnp.pad(b2, (0, c_pad - c))
    logits = _tc_linear(x, agg2[:, :n], w2p, b2p, relu=False)
    return logits[:, :c]


# speedup vs baseline: 3.6304x; 3.6232x over previous
import functools

import jax
import jax.numpy as jnp
from jax import lax
from jax.experimental import pallas as pl
from jax.experimental.pallas import tpu as pltpu
from jax.experimental.pallas import tpu_sc as plsc

_NC = 2
_NS = 16
_K = 80
_PH = 25


def _sc_segment_sum(x, e5, zeros_pad):
    n, d = x.shape
    n_phase = e5.shape[2]
    n_pad = zeros_pad.shape[0]
    rows_per_sub = n_pad // _NS
    assert e5.shape[3] == _PH and _PH >= 7

    mesh = plsc.VectorSubcoreMesh(core_axis_name="c", subcore_axis_name="s")

    @functools.partial(
        pl.kernel,
        out_type=jax.ShapeDtypeStruct((_NC, n_pad, d), jnp.float32),
        mesh=mesh,
        scratch_types=[
            pltpu.VMEM_SHARED((n_pad, d), jnp.float32),
            pltpu.VMEM((_PH, _K), jnp.int32),
            pltpu.VMEM((_PH, _K), jnp.int32),
            pltpu.VMEM((_K, d), jnp.float32),
            pltpu.VMEM((_K, d), jnp.float32),
            pltpu.VMEM((_K, d), jnp.float32),
            pltpu.SemaphoreType.DMA((3,)),
            pltpu.SemaphoreType.DMA((3,)),
        ],
    )
    def k(x_hbm, e_hbm, z_hbm, o_hbm, acc, sidx, didx, rows0, rows1, rows2,
          semg, sems):
        c = lax.axis_index("c")
        s = lax.axis_index("s")
        wid = c * _NS + s
        rbuf = (rows0, rows1, rows2)

        r0 = s * rows_per_sub
        pltpu.sync_copy(z_hbm.at[pl.ds(r0, rows_per_sub)],
                        acc.at[pl.ds(r0, rows_per_sub)])
        plsc.subcore_barrier()

        def g_start(b, i):
            pltpu.async_copy(x_hbm.at[sidx.at[b]], rbuf[i], semg.at[i])

        def g_wait(b, i):
            pltpu.make_async_copy(
                x_hbm.at[sidx.at[b]], rbuf[i], semg.at[i]).wait()

        def s_start(b, i):
            pltpu.async_copy(rbuf[i], acc.at[didx.at[b]], sems.at[i],
                             add=True)

        def s_wait(b, i):
            pltpu.make_async_copy(
                rbuf[i], acc.at[didx.at[b]], sems.at[i]).wait()

        def body(b, i):
            g_wait(b, i)
            s_start(b, i)
            s_wait(b - 1, (i - 1) % 3)
            g_start(b + 2, (i + 2) % 3)

        n_main = (_PH - 4) // 3
        for p in range(n_phase):
            pltpu.sync_copy(e_hbm.at[0, wid, p], sidx)
            pltpu.sync_copy(e_hbm.at[1, wid, p], didx)

            g_start(0, 0)
            g_start(1, 1)
            g_wait(0, 0)
            s_start(0, 0)
            g_start(2, 2)

            @pl.loop(0, n_main)
            def _(j):
                for i in (1, 2, 3):
                    body(3 * j + i, i % 3)

            for b in range(3 * n_main + 1, _PH - 2):
                body(b, b % 3)
            for b in (_PH - 2, _PH - 1):
                g_wait(b, b % 3)
                s_start(b, b % 3)
            for b in (_PH - 3, _PH - 2, _PH - 1):
                s_wait(b, b % 3)

        plsc.subcore_barrier()
        pltpu.sync_copy(acc.at[pl.ds(r0, rows_per_sub)],
                        o_hbm.at[c, pl.ds(r0, rows_per_sub)])

    return k(x, e5, zeros_pad)


def _tc_linear(x, agg, w, b, relu):
    n, d = x.shape
    h = w.shape[1]
    br = 1000

    def kern(x_ref, a_ref, w_ref, b_ref, o_ref):
        t = x_ref[...] + a_ref[0] + a_ref[1]
        y = jnp.dot(t, w_ref[...], preferred_element_type=jnp.float32)
        y = y + b_ref[...]
        o_ref[...] = jnp.maximum(y, 0.0) if relu else y

    return pl.pallas_call(
        kern,
        grid=(n // br,),
        in_specs=[
            pl.BlockSpec((br, d), lambda i: (i, 0)),
            pl.BlockSpec((_NC, br, d), lambda i: (0, i, 0)),
            pl.BlockSpec((d, h), lambda i: (0, 0)),
            pl.BlockSpec((1, h), lambda i: (0, 0)),
        ],
        out_specs=pl.BlockSpec((br, h), lambda i: (i, 0)),
        out_shape=jax.ShapeDtypeStruct((n, h), jnp.float32),
    )(x, agg, w, b.reshape(1, h))


def kernel(features, edge_index, W1, b1, W2, b2):
    n, d = features.shape
    e = edge_index.shape[1]
    c = W2.shape[1]
    n_workers = _NC * _NS
    n_per = e // n_workers
    n_phase = n_per // (_PH * _K)
    assert n_phase * _PH * _K == n_per
    n_pad = -(-n // 128) * 128

    e5 = edge_index.reshape(2, n_workers, n_phase, _PH, _K)
    zeros_pad = jnp.zeros((n_pad, d), jnp.float32)

    agg1 = _sc_segment_sum(features, e5, zeros_pad)
    x = _tc_linear(features, agg1[:, :n], W1, b1, relu=True)
    agg2 = _sc_segment_sum(x, e5, zeros_pad)

    c_pad = -(-c // 128) * 128
    w2p = jnp.pad(W2, ((0, 0), (0, c_pad - c)))
    b2p = jnp.pad(b2, (0, c_pad - c))
    logits = _tc_linear(x, agg2[:, :n], w2p, b2p, relu=False)
    return logits[:, :c]


# speedup vs baseline: 3.7276x; 1.0268x over previous
import functools

import jax
import jax.numpy as jnp
from jax import lax
from jax.experimental import pallas as pl
from jax.experimental.pallas import tpu as pltpu
from jax.experimental.pallas import tpu_sc as plsc

_NC = 2
_NS = 16
_K = 80
_PH = 25


def _sc_segment_sum(x, e5, zeros_pad):
    n, d = x.shape
    n_phase = e5.shape[2]
    n_pad = zeros_pad.shape[0]
    rows_per_sub = n_pad // _NS
    assert e5.shape[3] == _PH and _PH >= 7

    mesh = plsc.VectorSubcoreMesh(core_axis_name="c", subcore_axis_name="s")

    @functools.partial(
        pl.kernel,
        out_type=jax.ShapeDtypeStruct((_NC, n_pad, d), jnp.float32),
        mesh=mesh,
        scratch_types=[
            pltpu.VMEM_SHARED((n_pad, d), jnp.float32),
            pltpu.VMEM((_PH, _K), jnp.int32),
            pltpu.VMEM((_PH, _K), jnp.int32),
            pltpu.VMEM((_K, d), jnp.float32),
            pltpu.VMEM((_K, d), jnp.float32),
            pltpu.VMEM((_K, d), jnp.float32),
            pltpu.VMEM((_K, d), jnp.float32),
            pltpu.SemaphoreType.DMA((4,)),
            pltpu.SemaphoreType.DMA((4,)),
        ],
    )
    def k(x_hbm, e_hbm, z_hbm, o_hbm, acc, sidx, didx, rows0, rows1, rows2,
          rows3, semg, sems):
        c = lax.axis_index("c")
        s = lax.axis_index("s")
        wid = c * _NS + s
        rbuf = (rows0, rows1, rows2, rows3)

        r0 = s * rows_per_sub
        pltpu.sync_copy(z_hbm.at[pl.ds(r0, rows_per_sub)],
                        acc.at[pl.ds(r0, rows_per_sub)])
        plsc.subcore_barrier()

        def g_start(b, i):
            pltpu.async_copy(x_hbm.at[sidx.at[b]], rbuf[i], semg.at[i])

        def g_wait(b, i):
            pltpu.make_async_copy(
                x_hbm.at[sidx.at[b]], rbuf[i], semg.at[i]).wait()

        def s_start(b, i):
            pltpu.async_copy(rbuf[i], acc.at[didx.at[b]], sems.at[i],
                             add=True)

        def s_wait(b, i):
            pltpu.make_async_copy(
                rbuf[i], acc.at[didx.at[b]], sems.at[i]).wait()

        def body(b, i):
            g_wait(b, i)
            s_start(b, i)
            s_wait(b - 1, (i - 1) % 4)
            g_start(b + 3, (i + 3) % 4)

        n_main = (_PH - 5) // 4
        for p in range(n_phase):
            pltpu.sync_copy(e_hbm.at[0, wid, p], sidx)
            pltpu.sync_copy(e_hbm.at[1, wid, p], didx)

            g_start(0, 0)
            g_start(1, 1)
            g_start(2, 2)
            g_wait(0, 0)
            s_start(0, 0)
            g_start(3, 3)

            @pl.loop(0, n_main)
            def _(j):
                for i in (1, 2, 3, 4):
                    body(4 * j + i, i % 4)

            for b in range(4 * n_main + 1, _PH - 3):
                body(b, b % 4)
            for b in (_PH - 3, _PH - 2, _PH - 1):
                g_wait(b, b % 4)
                s_start(b, b % 4)
            for b in (_PH - 4, _PH - 3, _PH - 2, _PH - 1):
                s_wait(b, b % 4)

        plsc.subcore_barrier()
        pltpu.sync_copy(acc.at[pl.ds(r0, rows_per_sub)],
                        o_hbm.at[c, pl.ds(r0, rows_per_sub)])

    return k(x, e5, zeros_pad)


def _tc_linear(x, agg, w, b, relu):
    n, d = x.shape
    h = w.shape[1]
    br = 1000

    def kern(x_ref, a_ref, w_ref, b_ref, o_ref):
        t = x_ref[...] + a_ref[0] + a_ref[1]
        y = jnp.dot(t, w_ref[...], preferred_element_type=jnp.float32)
        y = y + b_ref[...]
        o_ref[...] = jnp.maximum(y, 0.0) if relu else y

    return pl.pallas_call(
        kern,
        grid=(n // br,),
        in_specs=[
            pl.BlockSpec((br, d), lambda i: (i, 0)),
            pl.BlockSpec((_NC, br, d), lambda i: (0, i, 0)),
            pl.BlockSpec((d, h), lambda i: (0, 0)),
            pl.BlockSpec((1, h), lambda i: (0, 0)),
        ],
        out_specs=pl.BlockSpec((br, h), lambda i: (i, 0)),
        out_shape=jax.ShapeDtypeStruct((n, h), jnp.float32),
    )(x, agg, w, b.reshape(1, h))


def kernel(features, edge_index, W1, b1, W2, b2):
    n, d = features.shape
    e = edge_index.shape[1]
    c = W2.shape[1]
    n_workers = _NC * _NS
    n_per = e // n_workers
    n_phase = n_per // (_PH * _K)
    assert n_phase * _PH * _K == n_per
    n_pad = -(-n // 128) * 128

    e5 = edge_index.reshape(2, n_workers, n_phase, _PH, _K)
    zeros_pad = jnp.zeros((n_pad, d), jnp.float32)

    agg1 = _sc_segment_sum(features, e5, zeros_pad)
    x = _tc_linear(features, agg1, W1, b1, relu=True)
    agg2 = _sc_segment_sum(x, e5, zeros_pad)

    c_pad = -(-c // 128) * 128
    w2p = jnp.pad(W2, ((0, 0), (0, c_pad - c)))
    b2p = jnp.pad(b2, (0, c_pad - c))
    logits = _tc_linear(x, agg2, w2p, b2p, relu=False)
    return logits[:, :c]


# speedup vs baseline: 3.7757x; 1.0129x over previous
import functools

import jax
import jax.numpy as jnp
from jax import lax
from jax.experimental import pallas as pl
from jax.experimental.pallas import tpu as pltpu
from jax.experimental.pallas import tpu_sc as plsc

_NC = 2
_NS = 16
_K = 80
_PH = 25


def _sc_segment_sum(x, e5, zeros_pad):
    n, d = x.shape
    n_phase = e5.shape[2]
    n_pad = zeros_pad.shape[0]
    rows_per_sub = n_pad // _NS
    assert e5.shape[3] == _PH and _PH >= 7

    mesh = plsc.VectorSubcoreMesh(core_axis_name="c", subcore_axis_name="s")

    @functools.partial(
        pl.kernel,
        out_type=jax.ShapeDtypeStruct((_NC, n_pad, d), jnp.float32),
        mesh=mesh,
        scratch_types=[
            pltpu.VMEM_SHARED((n_pad, d), jnp.float32),
            pltpu.VMEM((_PH, _K), jnp.int32),
            pltpu.VMEM((_PH, _K), jnp.int32),
            pltpu.VMEM((_K, d), jnp.float32),
            pltpu.VMEM((_K, d), jnp.float32),
            pltpu.VMEM((_K, d), jnp.float32),
            pltpu.VMEM((_K, d), jnp.float32),
            pltpu.SemaphoreType.DMA((4,)),
            pltpu.SemaphoreType.DMA((4,)),
        ],
    )
    def k(x_hbm, e_hbm, z_hbm, o_hbm, acc, sidx, didx, rows0, rows1, rows2,
          rows3, semg, sems):
        c = lax.axis_index("c")
        s = lax.axis_index("s")
        wid = c * _NS + s
        rbuf = (rows0, rows1, rows2, rows3)

        r0 = s * rows_per_sub

        def g_start(b, i):
            pltpu.async_copy(x_hbm.at[sidx.at[b]], rbuf[i], semg.at[i])

        def g_wait(b, i):
            pltpu.make_async_copy(
                x_hbm.at[sidx.at[b]], rbuf[i], semg.at[i]).wait()

        def s_start(b, i):
            pltpu.async_copy(rbuf[i], acc.at[didx.at[b]], sems.at[i],
                             add=True)

        def s_wait(b, i):
            pltpu.make_async_copy(
                rbuf[i], acc.at[didx.at[b]], sems.at[i]).wait()

        def body(b, i):
            g_wait(b, i)
            s_start(b, i)
            s_wait(b - 1, (i - 1) % 4)
            g_start(b + 3, (i + 3) % 4)

        n_main = (_PH - 5) // 4
        for p in range(n_phase):
            if p == 0:
                pltpu.sync_copy(e_hbm.at[0, wid, 0], sidx)
                pltpu.sync_copy(e_hbm.at[1, wid, 0], didx)
                g_start(0, 0)
                g_start(1, 1)
                g_start(2, 2)
                pltpu.sync_copy(z_hbm.at[pl.ds(r0, rows_per_sub)],
                                acc.at[pl.ds(r0, rows_per_sub)])
                plsc.subcore_barrier()
            else:
                pltpu.sync_copy(e_hbm.at[0, wid, p], sidx)
                pltpu.sync_copy(e_hbm.at[1, wid, p], didx)
                g_start(0, 0)
                g_start(1, 1)
                g_start(2, 2)
            g_wait(0, 0)
            s_start(0, 0)
            g_start(3, 3)

            @pl.loop(0, n_main)
            def _(j):
                for i in (1, 2, 3, 4):
                    body(4 * j + i, i % 4)

            for b in range(4 * n_main + 1, _PH - 3):
                body(b, b % 4)
            for b in (_PH - 3, _PH - 2, _PH - 1):
                g_wait(b, b % 4)
                s_start(b, b % 4)
            for b in (_PH - 4, _PH - 3, _PH - 2, _PH - 1):
                s_wait(b, b % 4)

        plsc.subcore_barrier()
        pltpu.sync_copy(acc.at[pl.ds(r0, rows_per_sub)],
                        o_hbm.at[c, pl.ds(r0, rows_per_sub)])

    return k(x, e5, zeros_pad)


def _tc_linear(x, agg, w, b, relu):
    n, d = x.shape
    h = w.shape[1]
    br = 1000

    def kern(x_ref, a_ref, w_ref, b_ref, o_ref):
        t = x_ref[...] + a_ref[0] + a_ref[1]
        y = jnp.dot(t, w_ref[...], preferred_element_type=jnp.float32)
        y = y + b_ref[...]
        o_ref[...] = jnp.maximum(y, 0.0) if relu else y

    return pl.pallas_call(
        kern,
        grid=(n // br,),
        in_specs=[
            pl.BlockSpec((br, d), lambda i: (i, 0)),
            pl.BlockSpec((_NC, br, d), lambda i: (0, i, 0)),
            pl.BlockSpec((d, h), lambda i: (0, 0)),
            pl.BlockSpec((1, h), lambda i: (0, 0)),
        ],
        out_specs=pl.BlockSpec((br, h), lambda i: (i, 0)),
        out_shape=jax.ShapeDtypeStruct((n, h), jnp.float32),
    )(x, agg, w, b.reshape(1, h))


def kernel(features, edge_index, W1, b1, W2, b2):
    n, d = features.shape
    e = edge_index.shape[1]
    c = W2.shape[1]
    n_workers = _NC * _NS
    n_per = e // n_workers
    n_phase = n_per // (_PH * _K)
    assert n_phase * _PH * _K == n_per
    n_pad = -(-n // 128) * 128

    e5 = edge_index.reshape(2, n_workers, n_phase, _PH, _K)
    zeros_pad = jnp.zeros((n_pad, d), jnp.float32)

    agg1 = _sc_segment_sum(features, e5, zeros_pad)
    x = _tc_linear(features, agg1, W1, b1, relu=True)
    agg2 = _sc_segment_sum(x, e5, zeros_pad)

    c_pad = -(-c // 128) * 128
    w2p = jnp.pad(W2, ((0, 0), (0, c_pad - c)))
    b2p = jnp.pad(b2, (0, c_pad - c))
    logits = _tc_linear(x, agg2, w2p, b2p, relu=False)
    return logits[:, :c]
